# Initial kernel scaffold; baseline (speedup 1.0000x reference)
import jax
import jax.numpy as jnp
from jax.experimental import pallas as pl


def kernel(nounMatrix, functor_table, context_table, X_argument, X_functor, X_context):
    raise NotImplementedError("write your pallas kernel here")


# speedup vs baseline: 2.7869x; 2.7869x over previous
import functools

import jax
import jax.numpy as jnp
from jax import lax
from jax.experimental import pallas as pl
from jax.experimental.pallas import tpu as pltpu
from jax.experimental.pallas import tpu_sc as plsc

EMBED = 32
BATCH = 16384
ROW = EMBED * EMBED
FOLD = 128 // EMBED
NOUN_BLOCKS = 100000 // FOLD
CTX_BLOCKS = 100000 // FOLD

NC = 2
NS = 16
NW = NC * NS
BPW = BATCH // NW
C = 32
NCHUNK = BPW // C
L = 16


def _splat(vec, i):
    idx = jnp.full((L, 1), i, jnp.int32)
    dn = lax.GatherDimensionNumbers(
        offset_dims=(), collapsed_slice_dims=(0,), start_index_map=(0,))
    return lax.gather(vec, idx, dn, (1,),
                      mode=lax.GatherScatterMode.PROMISE_IN_BOUNDS)


def _sc_body(noun_hbm, func_hbm, ctx_hbm, xa_hbm, xf_hbm, xc_hbm, out_hbm,
             idxa, idxf, idxc, idxa2, idxc2,
             a128, c128, abuf, cbuf, fbuf, pbuf, obuf, sem_f, sem_ac):
    wid = lax.axis_index("s") * NC + lax.axis_index("c")
    base = wid * BPW

    pltpu.sync_copy(xa_hbm.at[wid], idxa)
    pltpu.sync_copy(xf_hbm.at[wid], idxf)
    pltpu.sync_copy(xc_hbm.at[wid], idxc)

    def shift_body(k, _):
        s = pl.ds(k * L, L)
        idxa2[s] = lax.shift_right_logical(idxa[s], 2)
        idxc2[s] = lax.shift_right_logical(idxc[s], 2)
        return 0
    lax.fori_loop(0, BPW // L, shift_body, 0)

    lane = lax.iota(jnp.int32, L)

    def chunk_body(g, _):
        cs = pl.ds(g * C, C)
        cp_f = pltpu.make_async_copy(func_hbm.at[idxf.at[cs]], fbuf, sem_f)
        cp_a = pltpu.make_async_copy(noun_hbm.at[idxa2.at[cs]], a128, sem_ac)
        cp_c = pltpu.make_async_copy(ctx_hbm.at[idxc2.at[cs]], c128, sem_ac)
        cp_f.start(); cp_a.start(); cp_c.start()
        cp_a.wait(); cp_c.wait()

        for grp in range(C // L):
            row = grp * L + lane
            offa = (idxa[pl.ds(g * C + grp * L, L)] & 3) * EMBED
            offc = (idxc[pl.ds(g * C + grp * L, L)] & 3) * EMBED
            for j in range(EMBED):
                jv = jnp.full((L,), j, jnp.int32)
                va = plsc.load_gather(a128, [row, offa + jv])
                vc = plsc.load_gather(c128, [row, offc + jv])
                plsc.store_scatter(abuf, [row, jv], va)
                plsc.store_scatter(cbuf, [row, jv], vc)

        cp_f.wait()

        def body(b, _):
            a0 = abuf[b, pl.ds(0, L)]
            a1 = abuf[b, pl.ds(L, L)]
            c0 = cbuf[b, pl.ds(0, L)]
            c1 = cbuf[b, pl.ds(L, L)]
            w0 = jnp.zeros((L,), jnp.float32)
            w1 = jnp.zeros((L,), jnp.float32)
            for i in range(EMBED):
                cv = _splat(c0 if i < L else c1, i % L)
                w0 = w0 + cv * fbuf[b, pl.ds(i * EMBED, L)]
                w1 = w1 + cv * fbuf[b, pl.ds(i * EMBED + L, L)]
            pbuf[b, :] = w0 * a0 + w1 * a1
            return 0

        lax.fori_loop(0, C, body, 0)

        for grp in range(C // L):
            row_idx = grp * L + lane
            acc = jnp.zeros((L,), jnp.float32)
            for k in range(L):
                col_idx = jnp.full((L,), k, jnp.int32)
                acc = acc + plsc.load_gather(pbuf, [row_idx, col_idx])
            obuf[pl.ds(g * C + grp * L, L)] = acc
        return 0

    lax.fori_loop(0, NCHUNK, chunk_body, 0)

    pltpu.sync_copy(obuf, out_hbm.at[pl.ds(base, BPW)])


@jax.jit
def _sc_call(nounMatrix, functor_table, context_table, xa, xf, xc):
    mesh = plsc.VectorSubcoreMesh(core_axis_name="c", subcore_axis_name="s")
    f = pl.kernel(
        _sc_body,
        out_type=jax.ShapeDtypeStruct((BATCH,), jnp.float32),
        mesh=mesh,
        scratch_types=[
            pltpu.VMEM((BPW,), jnp.int32),
            pltpu.VMEM((BPW,), jnp.int32),
            pltpu.VMEM((BPW,), jnp.int32),
            pltpu.VMEM((BPW,), jnp.int32),
            pltpu.VMEM((BPW,), jnp.int32),
            pltpu.VMEM((C, 128), jnp.float32),
            pltpu.VMEM((C, 128), jnp.float32),
            pltpu.VMEM((C, EMBED), jnp.float32),
            pltpu.VMEM((C, EMBED), jnp.float32),
            pltpu.VMEM((C, ROW), jnp.float32),
            pltpu.VMEM((C, L), jnp.float32),
            pltpu.VMEM((BPW,), jnp.float32),
            pltpu.SemaphoreType.DMA,
            pltpu.SemaphoreType.DMA,
        ],
        compiler_params=pltpu.CompilerParams(needs_layout_passes=False),
    )
    return f(nounMatrix, functor_table, context_table, xa, xf, xc)


def kernel(nounMatrix, functor_table, context_table, X_argument, X_functor, X_context):
    noun128 = nounMatrix.reshape(NOUN_BLOCKS, 128)
    ctx128 = context_table.reshape(CTX_BLOCKS, 128)
    xa = X_argument.astype(jnp.int32).reshape(NW, BPW)
    xf = X_functor.astype(jnp.int32).reshape(NW, BPW)
    xc = X_context.astype(jnp.int32).reshape(NW, BPW)
    return _sc_call(noun128, functor_table, ctx128, xa, xf, xc)


# speedup vs baseline: 3.1083x; 1.1153x over previous
import functools

import jax
import jax.numpy as jnp
from jax import lax
from jax.experimental import pallas as pl
from jax.experimental.pallas import tpu as pltpu
from jax.experimental.pallas import tpu_sc as plsc

EMBED = 32
BATCH = 16384
ROW = EMBED * EMBED
FOLD = 128 // EMBED
NOUN_BLOCKS = 100000 // FOLD
CTX_BLOCKS = 100000 // FOLD

NC = 2
NS = 16
NW = NC * NS
BPW = BATCH // NW
C = 32
NCHUNK = BPW // C
L = 16


def _splat(vec, i):
    idx = jnp.full((L, 1), i, jnp.int32)
    dn = lax.GatherDimensionNumbers(
        offset_dims=(), collapsed_slice_dims=(0,), start_index_map=(0,))
    return lax.gather(vec, idx, dn, (1,),
                      mode=lax.GatherScatterMode.PROMISE_IN_BOUNDS)


def _sc_body(noun_hbm, func_hbm, ctx_hbm, xa_hbm, xf_hbm, xc_hbm, out_hbm,
             idxa, idxf, idxc, idxa2, idxc2,
             a128, c128, abuf, cbuf, fbuf0, fbuf1, pbuf, obuf,
             sem_f0, sem_f1, sem_ac0, sem_ac1):
    wid = lax.axis_index("s") * NC + lax.axis_index("c")
    base = wid * BPW

    pltpu.sync_copy(xa_hbm.at[wid], idxa)
    pltpu.sync_copy(xf_hbm.at[wid], idxf)
    pltpu.sync_copy(xc_hbm.at[wid], idxc)

    def shift_body(k, _):
        s = pl.ds(k * L, L)
        idxa2[s] = lax.shift_right_logical(idxa[s], 2)
        idxc2[s] = lax.shift_right_logical(idxc[s], 2)
        return 0
    lax.fori_loop(0, BPW // L, shift_body, 0)

    lane = lax.iota(jnp.int32, L)
    fbufs = (fbuf0, fbuf1)
    sems_f = (sem_f0, sem_f1)
    sems_ac = (sem_ac0, sem_ac1)

    def start_chunk(g, par):
        cs = pl.ds(g * C, C)
        a_sl = pl.ds(par * C, C)
        pltpu.make_async_copy(
            func_hbm.at[idxf.at[cs]], fbufs[par], sems_f[par]).start()
        pltpu.make_async_copy(
            noun_hbm.at[idxa2.at[cs]], a128.at[a_sl], sems_ac[par]).start()
        pltpu.make_async_copy(
            ctx_hbm.at[idxc2.at[cs]], c128.at[a_sl], sems_ac[par]).start()

    def wait_chunk(par):
        a_sl = pl.ds(par * C, C)
        pltpu.make_async_copy(
            func_hbm.at[idxf.at[pl.ds(0, C)]], fbufs[par], sems_f[par]).wait()
        pltpu.make_async_copy(
            noun_hbm.at[idxa2.at[pl.ds(0, C)]], a128.at[a_sl],
            sems_ac[par]).wait()
        pltpu.make_async_copy(
            ctx_hbm.at[idxc2.at[pl.ds(0, C)]], c128.at[a_sl],
            sems_ac[par]).wait()

    def run_chunk(g, par):
        fbuf = fbufs[par]
        wait_chunk(par)

        for grp in range(C // L):
            row = par * C + grp * L + lane
            orow = grp * L + lane
            offa = (idxa[pl.ds(g * C + grp * L, L)] & 3) * EMBED
            offc = (idxc[pl.ds(g * C + grp * L, L)] & 3) * EMBED
            for j in range(EMBED):
                jv = jnp.full((L,), j, jnp.int32)
                va = plsc.load_gather(a128, [row, offa + jv])
                vc = plsc.load_gather(c128, [row, offc + jv])
                plsc.store_scatter(abuf, [orow, jv], va)
                plsc.store_scatter(cbuf, [orow, jv], vc)

        def body(b, _):
            a0 = abuf[b, pl.ds(0, L)]
            a1 = abuf[b, pl.ds(L, L)]
            c0 = cbuf[b, pl.ds(0, L)]
            c1 = cbuf[b, pl.ds(L, L)]
            w0 = jnp.zeros((L,), jnp.float32)
            w1 = jnp.zeros((L,), jnp.float32)
            for i in range(EMBED):
                cv = _splat(c0 if i < L else c1, i % L)
                w0 = w0 + cv * fbuf[b, pl.ds(i * EMBED, L)]
                w1 = w1 + cv * fbuf[b, pl.ds(i * EMBED + L, L)]
            pbuf[b, :] = w0 * a0 + w1 * a1
            return 0

        lax.fori_loop(0, C, body, 0)

        for grp in range(C // L):
            row_idx = grp * L + lane
            acc = jnp.zeros((L,), jnp.float32)
            for k in range(L):
                col_idx = jnp.full((L,), k, jnp.int32)
                acc = acc + plsc.load_gather(pbuf, [row_idx, col_idx])
            obuf[pl.ds(g * C + grp * L, L)] = acc

    start_chunk(0, 0)

    def pair_body(h, _):
        for par in range(2):
            g = 2 * h + par
            nxt = g + 1

            @pl.when(nxt < NCHUNK)
            def _():
                start_chunk(nxt, 1 - par)

            run_chunk(g, par)
        return 0

    lax.fori_loop(0, NCHUNK // 2, pair_body, 0)

    pltpu.sync_copy(obuf, out_hbm.at[pl.ds(base, BPW)])


@jax.jit
def _sc_call(nounMatrix, functor_table, context_table, xa, xf, xc):
    mesh = plsc.VectorSubcoreMesh(core_axis_name="c", subcore_axis_name="s")
    f = pl.kernel(
        _sc_body,
        out_type=jax.ShapeDtypeStruct((BATCH,), jnp.float32),
        mesh=mesh,
        scratch_types=[
            pltpu.VMEM((BPW,), jnp.int32),
            pltpu.VMEM((BPW,), jnp.int32),
            pltpu.VMEM((BPW,), jnp.int32),
            pltpu.VMEM((BPW,), jnp.int32),
            pltpu.VMEM((BPW,), jnp.int32),
            pltpu.VMEM((2 * C, 128), jnp.float32),
            pltpu.VMEM((2 * C, 128), jnp.float32),
            pltpu.VMEM((C, EMBED), jnp.float32),
            pltpu.VMEM((C, EMBED), jnp.float32),
            pltpu.VMEM((C, ROW), jnp.float32),
            pltpu.VMEM((C, ROW), jnp.float32),
            pltpu.VMEM((C, L), jnp.float32),
            pltpu.VMEM((BPW,), jnp.float32),
            pltpu.SemaphoreType.DMA,
            pltpu.SemaphoreType.DMA,
            pltpu.SemaphoreType.DMA,
            pltpu.SemaphoreType.DMA,
        ],
        compiler_params=pltpu.CompilerParams(needs_layout_passes=False),
    )
    return f(nounMatrix, functor_table, context_table, xa, xf, xc)


def kernel(nounMatrix, functor_table, context_table, X_argument, X_functor, X_context):
    noun128 = nounMatrix.reshape(NOUN_BLOCKS, 128)
    ctx128 = context_table.reshape(CTX_BLOCKS, 128)
    xa = X_argument.astype(jnp.int32).reshape(NW, BPW)
    xf = X_functor.astype(jnp.int32).reshape(NW, BPW)
    xc = X_context.astype(jnp.int32).reshape(NW, BPW)
    return _sc_call(noun128, functor_table, ctx128, xa, xf, xc)


# speedup vs baseline: 3.7810x; 1.2164x over previous
import functools

import jax
import jax.numpy as jnp
from jax import lax
from jax.experimental import pallas as pl
from jax.experimental.pallas import tpu as pltpu
from jax.experimental.pallas import tpu_sc as plsc

EMBED = 32
BATCH = 16384
ROW = EMBED * EMBED

NC = 2
NS = 16
NW = NC * NS
BPW = BATCH // NW
C = 32
NCHUNK = BPW // C
L = 16


def _splat(vec, i):
    idx = jnp.full((L, 1), i, jnp.int32)
    dn = lax.GatherDimensionNumbers(
        offset_dims=(), collapsed_slice_dims=(0,), start_index_map=(0,))
    return lax.gather(vec, idx, dn, (1,),
                      mode=lax.GatherScatterMode.PROMISE_IN_BOUNDS)


def _sc_body(noun_hbm, func_hbm, ctx_hbm, xa_hbm, xf_hbm, xc_hbm, out_hbm,
             idxa, idxf, idxc,
             a128, c128, fbuf0, fbuf1, pbuf, obuf,
             sem_f0, sem_f1, sem_ac0, sem_ac1):
    wid = lax.axis_index("s") * NC + lax.axis_index("c")
    base = wid * BPW

    pltpu.sync_copy(xa_hbm.at[wid], idxa)
    pltpu.sync_copy(xf_hbm.at[wid], idxf)
    pltpu.sync_copy(xc_hbm.at[wid], idxc)

    lane = lax.iota(jnp.int32, L)
    fbufs = (fbuf0, fbuf1)
    sems_f = (sem_f0, sem_f1)
    sems_ac = (sem_ac0, sem_ac1)

    def start_chunk(g, par):
        cs = pl.ds(g * C, C)
        a_sl = pl.ds(par * C, C)
        pltpu.make_async_copy(
            func_hbm.at[idxf.at[cs]], fbufs[par], sems_f[par]).start()
        pltpu.make_async_copy(
            noun_hbm.at[idxa.at[cs]], a128.at[a_sl], sems_ac[par]).start()
        pltpu.make_async_copy(
            ctx_hbm.at[idxc.at[cs]], c128.at[a_sl], sems_ac[par]).start()

    def wait_chunk(par):
        a_sl = pl.ds(par * C, C)
        pltpu.make_async_copy(
            func_hbm.at[idxf.at[pl.ds(0, C)]], fbufs[par], sems_f[par]).wait()
        pltpu.make_async_copy(
            noun_hbm.at[idxa.at[pl.ds(0, C)]], a128.at[a_sl],
            sems_ac[par]).wait()
        pltpu.make_async_copy(
            ctx_hbm.at[idxc.at[pl.ds(0, C)]], c128.at[a_sl],
            sems_ac[par]).wait()

    def run_chunk(g, par):
        fbuf = fbufs[par]
        wait_chunk(par)

        def body(b, _):
            br = par * C + b
            a0 = a128[br, pl.ds(0, L)]
            a1 = a128[br, pl.ds(L, L)]
            c0 = c128[br, pl.ds(0, L)]
            c1 = c128[br, pl.ds(L, L)]
            w0 = jnp.zeros((L,), jnp.float32)
            w1 = jnp.zeros((L,), jnp.float32)
            for i in range(EMBED):
                cv = _splat(c0 if i < L else c1, i % L)
                w0 = w0 + cv * fbuf[b, pl.ds(i * EMBED, L)]
                w1 = w1 + cv * fbuf[b, pl.ds(i * EMBED + L, L)]
            pbuf[b, :] = w0 * a0 + w1 * a1
            return 0

        lax.fori_loop(0, C, body, 0)

        for grp in range(C // L):
            row_idx = grp * L + lane
            acc = jnp.zeros((L,), jnp.float32)
            for k in range(L):
                col_idx = jnp.full((L,), k, jnp.int32)
                acc = acc + plsc.load_gather(pbuf, [row_idx, col_idx])
            obuf[pl.ds(g * C + grp * L, L)] = acc

    start_chunk(0, 0)

    def pair_body(h, _):
        for par in range(2):
            g = 2 * h + par
            nxt = g + 1

            @pl.when(nxt < NCHUNK)
            def _():
                start_chunk(nxt, 1 - par)

            run_chunk(g, par)
        return 0

    lax.fori_loop(0, NCHUNK // 2, pair_body, 0)

    pltpu.sync_copy(obuf, out_hbm.at[pl.ds(base, BPW)])


@jax.jit
def _sc_call(noun128, functor_table, ctx128, xa, xf, xc):
    mesh = plsc.VectorSubcoreMesh(core_axis_name="c", subcore_axis_name="s")
    f = pl.kernel(
        _sc_body,
        out_type=jax.ShapeDtypeStruct((BATCH,), jnp.float32),
        mesh=mesh,
        scratch_types=[
            pltpu.VMEM((BPW,), jnp.int32),
            pltpu.VMEM((BPW,), jnp.int32),
            pltpu.VMEM((BPW,), jnp.int32),
            pltpu.VMEM((2 * C, 128), jnp.float32),
            pltpu.VMEM((2 * C, 128), jnp.float32),
            pltpu.VMEM((C, ROW), jnp.float32),
            pltpu.VMEM((C, ROW), jnp.float32),
            pltpu.VMEM((C, L), jnp.float32),
            pltpu.VMEM((BPW,), jnp.float32),
            pltpu.SemaphoreType.DMA,
            pltpu.SemaphoreType.DMA,
            pltpu.SemaphoreType.DMA,
            pltpu.SemaphoreType.DMA,
        ],
        compiler_params=pltpu.CompilerParams(needs_layout_passes=False),
    )
    return f(noun128, functor_table, ctx128, xa, xf, xc)


def kernel(nounMatrix, functor_table, context_table, X_argument, X_functor, X_context):
    noun128 = jnp.pad(nounMatrix, ((0, 0), (0, 128 - EMBED)))
    ctx128 = jnp.pad(context_table, ((0, 0), (0, 128 - EMBED)))
    xa = X_argument.astype(jnp.int32).reshape(NW, BPW)
    xf = X_functor.astype(jnp.int32).reshape(NW, BPW)
    xc = X_context.astype(jnp.int32).reshape(NW, BPW)
    return _sc_call(noun128, functor_table, ctx128, xa, xf, xc)


# speedup vs baseline: 4.0160x; 1.0622x over previous
import functools

import jax
import jax.numpy as jnp
from jax import lax
from jax.experimental import pallas as pl
from jax.experimental.pallas import tpu as pltpu
from jax.experimental.pallas import tpu_sc as plsc

EMBED = 32
BATCH = 16384
ROW = EMBED * EMBED

NC = 2
NS = 16
NW = NC * NS
BPW = BATCH // NW
C = 32
NCHUNK = BPW // C
L = 16


def _splat(vec, i):
    idx = jnp.full((L, 1), i, jnp.int32)
    dn = lax.GatherDimensionNumbers(
        offset_dims=(), collapsed_slice_dims=(0,), start_index_map=(0,))
    return lax.gather(vec, idx, dn, (1,),
                      mode=lax.GatherScatterMode.PROMISE_IN_BOUNDS)


def _sc_body(noun_hbm, func_hbm, ctx_hbm, xa_hbm, xf_hbm, xc_hbm, out_hbm,
             idxa, idxf, idxc,
             a128, c128, fbuf0, fbuf1, pbuf, obuf,
             sem_f0, sem_f1, sem_ac0, sem_ac1):
    wid = lax.axis_index("s") * NC + lax.axis_index("c")
    base = wid * BPW

    pltpu.sync_copy(xa_hbm.at[wid], idxa)
    pltpu.sync_copy(xf_hbm.at[wid], idxf)
    pltpu.sync_copy(xc_hbm.at[wid], idxc)

    lane = lax.iota(jnp.int32, L)
    fbufs = (fbuf0, fbuf1)
    sems_f = (sem_f0, sem_f1)
    sems_ac = (sem_ac0, sem_ac1)

    def start_chunk(g, par):
        cs = pl.ds(g * C, C)
        a_sl = pl.ds(par * C, C)
        pltpu.make_async_copy(
            func_hbm.at[idxf.at[cs]], fbufs[par], sems_f[par]).start()
        pltpu.make_async_copy(
            noun_hbm.at[idxa.at[cs]], a128.at[a_sl], sems_ac[par]).start()
        pltpu.make_async_copy(
            ctx_hbm.at[idxc.at[cs]], c128.at[a_sl], sems_ac[par]).start()

    def wait_chunk(par):
        a_sl = pl.ds(par * C, C)
        pltpu.make_async_copy(
            func_hbm.at[idxf.at[pl.ds(0, C)]], fbufs[par], sems_f[par]).wait()
        pltpu.make_async_copy(
            noun_hbm.at[idxa.at[pl.ds(0, C)]], a128.at[a_sl],
            sems_ac[par]).wait()
        pltpu.make_async_copy(
            ctx_hbm.at[idxc.at[pl.ds(0, C)]], c128.at[a_sl],
            sems_ac[par]).wait()

    def run_chunk(g, par):
        fbuf = fbufs[par]
        wait_chunk(par)

        def body(b, _):
            br = par * C + b
            a0 = a128[br, pl.ds(0, L)]
            a1 = a128[br, pl.ds(L, L)]
            c0 = c128[br, pl.ds(EMBED, L)]
            c1 = c128[br, pl.ds(EMBED + L, L)]
            w0 = jnp.zeros((L,), jnp.float32)
            w1 = jnp.zeros((L,), jnp.float32)
            for i in range(EMBED):
                cv = _splat(c0 if i < L else c1, i % L)
                w0 = w0 + cv * fbuf[b, pl.ds(i * EMBED, L)]
                w1 = w1 + cv * fbuf[b, pl.ds(i * EMBED + L, L)]
            pbuf[b, :] = w0 * a0 + w1 * a1
            return 0

        lax.fori_loop(0, C, body, 0)

        for grp in range(C // L):
            row_idx = grp * L + lane
            acc = jnp.zeros((L,), jnp.float32)
            for k in range(L):
                col_idx = jnp.full((L,), k, jnp.int32)
                acc = acc + plsc.load_gather(pbuf, [row_idx, col_idx])
            obuf[pl.ds(g * C + grp * L, L)] = acc

    start_chunk(0, 0)

    def pair_body(h, _):
        for par in range(2):
            g = 2 * h + par
            nxt = g + 1

            @pl.when(nxt < NCHUNK)
            def _():
                start_chunk(nxt, 1 - par)

            run_chunk(g, par)
        return 0

    lax.fori_loop(0, NCHUNK // 2, pair_body, 0)

    pltpu.sync_copy(obuf, out_hbm.at[pl.ds(base, BPW)])


@jax.jit
def _sc_call(noun128, functor_table, ctx128, xa, xf, xc):
    mesh = plsc.VectorSubcoreMesh(core_axis_name="c", subcore_axis_name="s")
    f = pl.kernel(
        _sc_body,
        out_type=jax.ShapeDtypeStruct((BATCH,), jnp.float32),
        mesh=mesh,
        scratch_types=[
            pltpu.VMEM((BPW,), jnp.int32),
            pltpu.VMEM((BPW,), jnp.int32),
            pltpu.VMEM((BPW,), jnp.int32),
            pltpu.VMEM((2 * C, 128), jnp.float32),
            pltpu.VMEM((2 * C, 128), jnp.float32),
            pltpu.VMEM((C, ROW), jnp.float32),
            pltpu.VMEM((C, ROW), jnp.float32),
            pltpu.VMEM((C, L), jnp.float32),
            pltpu.VMEM((BPW,), jnp.float32),
            pltpu.SemaphoreType.DMA,
            pltpu.SemaphoreType.DMA,
            pltpu.SemaphoreType.DMA,
            pltpu.SemaphoreType.DMA,
        ],
        compiler_params=pltpu.CompilerParams(needs_layout_passes=False),
    )
    return f(noun128, functor_table, ctx128, xa, xf, xc)


def kernel(nounMatrix, functor_table, context_table, X_argument, X_functor, X_context):
    combo = jnp.concatenate(
        [nounMatrix, context_table,
         jnp.zeros((100000, 128 - 2 * EMBED), jnp.float32)], axis=1)
    xa = X_argument.astype(jnp.int32).reshape(NW, BPW)
    xf = X_functor.astype(jnp.int32).reshape(NW, BPW)
    xc = X_context.astype(jnp.int32).reshape(NW, BPW)
    return _sc_call(combo, functor_table, combo, xa, xf, xc)
